# baseline (speedup 1.0000x reference)
import numpy as np
import jax
import jax.numpy as jnp
from jax import lax
from jax.experimental import pallas as pl
from jax.experimental.pallas import tpu as pltpu
from jax.experimental.pallas import tpu_sc as plsc

_SPAN_MAX_LEN = 16
_BINS = (0, 1, 2, 3, 4, 5, 7, 8, 15, 16, 31, 32, 63, 64)
_B, _S, _D = 4, 512, 768
_E = 64
_ROW = 2 * _D + _E
_N = sum(_S - w + 1 for w in range(1, _SPAN_MAX_LEN + 1))
_LN = 128
_NCHUNK = _S // _LN
_NCB = _D // _LN
_NC, _NS = 2, 16
_TASKS_PER_WORKER = (_B * _SPAN_MAX_LEN) // (_NC * _NS)


def _win_off(w):
    return (_S + 1) * (w - 1) - ((w - 1) * w) // 2


def _bucket(w):
    bk = jnp.int32(-1)
    for bn in _BINS:
        bk = bk + (w >= bn).astype(jnp.int32)
    return bk


def _body(x_hbm, emb_hbm, out_hbm,
          cstage, cbuf, ebuf, svec, evec, bvec, etab, wsem, esem):
    cid = lax.axis_index("c")
    sid = lax.axis_index("s")
    wid = sid * _NC + cid

    pltpu.sync_copy(emb_hbm, etab)

    iota16 = jnp.arange(16, dtype=jnp.int32)

    def wait_slot(buf, sem, sl, nrows):
        pltpu.make_async_copy(
            buf[sl],
            out_hbm.at[0, pl.ds(0, nrows), pl.ds(0, _LN)],
            sem[sl]).wait()

    for t in range(_TASKS_PER_WORKER):
        tid = wid * _TASKS_PER_WORKER + t
        b = tid // _SPAN_MAX_LEN
        w = tid % _SPAN_MAX_LEN + 1
        off = _win_off(w)
        off_next = _win_off(w + 1)
        bk1 = _bucket(w)
        bk2 = _bucket(w + 1)
        n_base = (w - 1) * _S

        for j in range(_NCHUNK):
            for g in range(8):
                n = n_base + j * _LN + g * 16 + iota16
                in2 = n >= off_next
                s = n - jnp.where(in2, off_next, off)
                e = s + jnp.where(in2, w, w - 1)
                o = j * _LN + g * 16
                svec[pl.ds(o, 16)] = s
                evec[pl.ds(o, 16)] = e
                bvec[pl.ds(o, 16)] = jnp.where(in2, bk2, bk1)

        def cb_body(cb, carry):
            pltpu.sync_copy(
                x_hbm.at[pl.ds(b * _S, _S), pl.ds(cb * _LN, _LN)], cstage)
            for role in range(2):
                vec = svec if role == 0 else evec

                def jj_body(jj, carry2, role=role):
                    for half in range(2):
                        j = jj * 2 + half
                        sl = half
                        if t == 0 and role == 0:
                            @pl.when((cb > 0) | (jj > 0))
                            def _():
                                wait_slot(cbuf, wsem, sl, _LN)
                        else:
                            wait_slot(cbuf, wsem, sl, _LN)
                        rows = [vec[pl.ds(j * _LN + g * 16, 16)]
                                for g in range(8)]

                        @plsc.parallel_loop(0, _LN, 1, unroll=16)
                        def _fill(c, rows=rows, sl=sl):
                            cc = jnp.full((16,), c, dtype=jnp.int32)
                            for g in range(8):
                                v = plsc.load_gather(cstage, [rows[g], cc])
                                cbuf[sl][c, pl.ds(g * 16, 16)] = v
                        dst = out_hbm.at[b,
                                         pl.ds(role * _D + cb * _LN, _LN),
                                         pl.ds(n_base + j * _LN, _LN)]
                        pltpu.async_copy(cbuf[sl], dst, wsem[sl])
                    return carry2

                lax.fori_loop(0, _NCHUNK // 2, jj_body, 0)
            return carry

        lax.fori_loop(0, _NCB, cb_body, 0)

        def ej_body(jj, carry2):
            for half in range(2):
                j = jj * 2 + half
                sl = half
                if t == 0:
                    @pl.when(jj > 0)
                    def _():
                        wait_slot(ebuf, esem, sl, _E)
                else:
                    wait_slot(ebuf, esem, sl, _E)
                bks = [bvec[pl.ds(j * _LN + g * 16, 16)] for g in range(8)]

                @plsc.parallel_loop(0, _E, 1, unroll=16)
                def _fill_e(c, bks=bks, sl=sl):
                    cc = jnp.full((16,), c, dtype=jnp.int32)
                    for g in range(8):
                        v = plsc.load_gather(etab, [bks[g], cc])
                        ebuf[sl][c, pl.ds(g * 16, 16)] = v
                dst = out_hbm.at[b,
                                 pl.ds(2 * _D, _E),
                                 pl.ds(n_base + j * _LN, _LN)]
                pltpu.async_copy(ebuf[sl], dst, esem[sl])
            return carry2

        lax.fori_loop(0, _NCHUNK // 2, ej_body, 0)

    for sl in (0, 1):
        wait_slot(cbuf, wsem, sl, _LN)
        wait_slot(ebuf, esem, sl, _E)


def _span_index_table():
    starts_list, ends_list = [], []
    for w in range(1, _SPAN_MAX_LEN + 1):
        st = np.arange(0, _S - w + 1, dtype=np.int32)
        starts_list.append(st)
        ends_list.append(st + w - 1)
    return np.concatenate(starts_list), np.concatenate(ends_list)


_STARTS_NP, _ENDS_NP = _span_index_table()


def kernel(x, emb_table, batch_max_seq_len):
    mesh = plsc.VectorSubcoreMesh(core_axis_name="c", subcore_axis_name="s")
    out_t = pl.kernel(
        _body,
        mesh=mesh,
        compiler_params=pltpu.CompilerParams(needs_layout_passes=False),
        out_type=jax.ShapeDtypeStruct((_B, _ROW, _N), jnp.float32),
        scratch_types=[
            pltpu.VMEM((_S, _LN), jnp.float32),
            [pltpu.VMEM((_LN, _LN), jnp.float32)] * 2,
            [pltpu.VMEM((_E, _LN), jnp.float32)] * 2,
            pltpu.VMEM((_NCHUNK * _LN,), jnp.int32),
            pltpu.VMEM((_NCHUNK * _LN,), jnp.int32),
            pltpu.VMEM((_NCHUNK * _LN,), jnp.int32),
            pltpu.VMEM((len(_BINS), _E), jnp.float32),
            [pltpu.SemaphoreType.DMA] * 2,
            [pltpu.SemaphoreType.DMA] * 2,
        ],
    )(x.reshape(_B * _S, _D), emb_table)
    out = jnp.transpose(out_t, (0, 2, 1))

    starts_j = jnp.asarray(_STARTS_NP)
    ends_j = jnp.minimum(jnp.asarray(_ENDS_NP), batch_max_seq_len - 1)
    span_indices = jnp.stack([starts_j, ends_j], axis=1)
    return out, span_indices


# speedup vs baseline: 1.0065x; 1.0065x over previous
import numpy as np
import jax
import jax.numpy as jnp
from jax import lax
from jax.experimental import pallas as pl
from jax.experimental.pallas import tpu as pltpu
from jax.experimental.pallas import tpu_sc as plsc

_SPAN_MAX_LEN = 16
_BINS = (0, 1, 2, 3, 4, 5, 7, 8, 15, 16, 31, 32, 63, 64)
_B, _S, _D = 4, 512, 768
_E = 64
_ROW = 2 * _D + _E
_N = sum(_S - w + 1 for w in range(1, _SPAN_MAX_LEN + 1))
_LN = 128
_NCHUNK = _S // _LN
_NCB = _D // _LN
_NC, _NS = 2, 16
_TASKS_PER_WORKER = (_B * _SPAN_MAX_LEN) // (_NC * _NS)


def _win_off(w):
    return (_S + 1) * (w - 1) - ((w - 1) * w) // 2


def _bucket(w):
    bk = jnp.int32(-1)
    for bn in _BINS:
        bk = bk + (w >= bn).astype(jnp.int32)
    return bk


def _body(x_hbm, emb_hbm, out_hbm,
          cstage, cbuf, ebuf, svec, evec, bvec, etab, wsem, esem):
    cid = lax.axis_index("c")
    sid = lax.axis_index("s")
    wid = sid * _NC + cid

    pltpu.sync_copy(emb_hbm, etab)

    iota16 = jnp.arange(16, dtype=jnp.int32)

    def wait_slot(buf, sem, sl, nrows):
        pltpu.make_async_copy(
            buf[sl],
            out_hbm.at[0, pl.ds(0, nrows), pl.ds(0, _LN)],
            sem[sl]).wait()

    for t in range(_TASKS_PER_WORKER):
        tid = wid * _TASKS_PER_WORKER + t
        b = tid // _SPAN_MAX_LEN
        w = tid % _SPAN_MAX_LEN + 1
        off = _win_off(w)
        off_next = _win_off(w + 1)
        bk1 = _bucket(w)
        bk2 = _bucket(w + 1)
        n_base = (w - 1) * _S

        for j in range(_NCHUNK):
            for g in range(8):
                n = n_base + j * _LN + g * 16 + iota16
                in2 = n >= off_next
                s = n - jnp.where(in2, off_next, off)
                e = s + jnp.where(in2, w, w - 1)
                o = j * _LN + g * 16
                svec[pl.ds(o, 16)] = s
                evec[pl.ds(o, 16)] = e
                bvec[pl.ds(o, 16)] = jnp.where(in2, bk2, bk1)

        def cb_body(cb, carry):
            pltpu.sync_copy(
                x_hbm.at[pl.ds(b * _S, _S), pl.ds(cb * _LN, _LN)], cstage)
            for role in range(2):
                vec = svec if role == 0 else evec

                def jj_body(jj, carry2, role=role):
                    for half in range(2):
                        j = jj * 2 + half
                        sl = half
                        if t == 0 and role == 0:
                            @pl.when((cb > 0) | (jj > 0))
                            def _():
                                wait_slot(cbuf, wsem, sl, _LN)
                        else:
                            wait_slot(cbuf, wsem, sl, _LN)
                        rows = [vec[pl.ds(j * _LN + g * 16, 16)]
                                for g in range(8)]

                        @plsc.parallel_loop(0, _LN, 1, unroll=4)
                        def _fill(c, rows=rows, sl=sl):
                            cc = jnp.full((16,), c, dtype=jnp.int32)
                            for g in range(8):
                                v = plsc.load_gather(cstage, [rows[g], cc])
                                cbuf[sl][c, pl.ds(g * 16, 16)] = v
                        dst = out_hbm.at[b,
                                         pl.ds(role * _D + cb * _LN, _LN),
                                         pl.ds(n_base + j * _LN, _LN)]
                        pltpu.async_copy(cbuf[sl], dst, wsem[sl])
                    return carry2

                lax.fori_loop(0, _NCHUNK // 2, jj_body, 0)
            return carry

        lax.fori_loop(0, _NCB, cb_body, 0)

        def ej_body(jj, carry2):
            for half in range(2):
                j = jj * 2 + half
                sl = half
                if t == 0:
                    @pl.when(jj > 0)
                    def _():
                        wait_slot(ebuf, esem, sl, _E)
                else:
                    wait_slot(ebuf, esem, sl, _E)
                bks = [bvec[pl.ds(j * _LN + g * 16, 16)] for g in range(8)]

                @plsc.parallel_loop(0, _E, 1, unroll=4)
                def _fill_e(c, bks=bks, sl=sl):
                    cc = jnp.full((16,), c, dtype=jnp.int32)
                    for g in range(8):
                        v = plsc.load_gather(etab, [bks[g], cc])
                        ebuf[sl][c, pl.ds(g * 16, 16)] = v
                dst = out_hbm.at[b,
                                 pl.ds(2 * _D, _E),
                                 pl.ds(n_base + j * _LN, _LN)]
                pltpu.async_copy(ebuf[sl], dst, esem[sl])
            return carry2

        lax.fori_loop(0, _NCHUNK // 2, ej_body, 0)

    for sl in (0, 1):
        wait_slot(cbuf, wsem, sl, _LN)
        wait_slot(ebuf, esem, sl, _E)


def _span_index_table():
    starts_list, ends_list = [], []
    for w in range(1, _SPAN_MAX_LEN + 1):
        st = np.arange(0, _S - w + 1, dtype=np.int32)
        starts_list.append(st)
        ends_list.append(st + w - 1)
    return np.concatenate(starts_list), np.concatenate(ends_list)


_STARTS_NP, _ENDS_NP = _span_index_table()


def kernel(x, emb_table, batch_max_seq_len):
    mesh = plsc.VectorSubcoreMesh(core_axis_name="c", subcore_axis_name="s")
    out_t = pl.kernel(
        _body,
        mesh=mesh,
        compiler_params=pltpu.CompilerParams(needs_layout_passes=False),
        out_type=jax.ShapeDtypeStruct((_B, _ROW, _N), jnp.float32),
        scratch_types=[
            pltpu.VMEM((_S, _LN), jnp.float32),
            [pltpu.VMEM((_LN, _LN), jnp.float32)] * 2,
            [pltpu.VMEM((_E, _LN), jnp.float32)] * 2,
            pltpu.VMEM((_NCHUNK * _LN,), jnp.int32),
            pltpu.VMEM((_NCHUNK * _LN,), jnp.int32),
            pltpu.VMEM((_NCHUNK * _LN,), jnp.int32),
            pltpu.VMEM((len(_BINS), _E), jnp.float32),
            [pltpu.SemaphoreType.DMA] * 2,
            [pltpu.SemaphoreType.DMA] * 2,
        ],
    )(x.reshape(_B * _S, _D), emb_table)
    out = jnp.transpose(out_t, (0, 2, 1))

    starts_j = jnp.asarray(_STARTS_NP)
    ends_j = jnp.minimum(jnp.asarray(_ENDS_NP), batch_max_seq_len - 1)
    span_indices = jnp.stack([starts_j, ends_j], axis=1)
    return out, span_indices


# speedup vs baseline: 1.7874x; 1.7758x over previous
import numpy as np
import jax
import jax.numpy as jnp
from jax import lax
from jax.experimental import pallas as pl
from jax.experimental.pallas import tpu as pltpu
from jax.experimental.pallas import tpu_sc as plsc

_SPAN_MAX_LEN = 16
_BINS = (0, 1, 2, 3, 4, 5, 7, 8, 15, 16, 31, 32, 63, 64)
_B, _S, _D = 4, 512, 768
_E = 64
_ROW = 2 * _D + _E
_N = sum(_S - w + 1 for w in range(1, _SPAN_MAX_LEN + 1))
_CH = 32
_NCHUNK = _S // _CH
_NC, _NS = 2, 16
_TASKS_PER_WORKER = (_B * _SPAN_MAX_LEN) // (_NC * _NS)


def _win_off(w):
    return (_S + 1) * (w - 1) - ((w - 1) * w) // 2


def _bucket(w):
    bk = jnp.int32(-1)
    for bn in _BINS:
        bk = bk + (w >= bn).astype(jnp.int32)
    return bk


def _body(x_hbm, emb_hbm, out_hbm,
          rowbuf, sidx, eidx, etab, gsem, wsem):
    cid = lax.axis_index("c")
    sid = lax.axis_index("s")
    wid = sid * _NC + cid

    pltpu.sync_copy(emb_hbm, etab)

    for t in range(_TASKS_PER_WORKER):
        tid = wid * _TASKS_PER_WORKER + t
        b = tid // _SPAN_MAX_LEN
        w = tid % _SPAN_MAX_LEN + 1
        off = _win_off(w)
        off_next = _win_off(w + 1)
        bk1 = _bucket(w)
        bk2 = _bucket(w + 1)
        r_lo = (off + 7) // 8 * 8
        r_hi = (off_next + 7) // 8 * 8
        xbase = b * _S

        def chunk_row(cix):
            return jnp.minimum(r_lo + cix * _CH, r_hi - _CH)

        def build_idx(cix, sl):
            r0 = chunk_row(cix)
            for k in range(_CH // 16):
                n = r0 + (k * 16 + jnp.arange(16, dtype=jnp.int32))
                in2 = n >= off_next
                s = n - jnp.where(in2, off_next, off)
                e = s + jnp.where(in2, w, w - 1)
                sidx[sl][pl.ds(k * 16, 16)] = s + xbase
                eidx[sl][pl.ds(k * 16, 16)] = e + xbase

            def _fill(i, carry):
                bk = jnp.where(r0 + i >= off_next, bk2, bk1)
                for k in range(_E // 16):
                    rowbuf[sl][i, pl.ds(2 * _D + k * 16, 16)] = \
                        etab[bk, pl.ds(k * 16, 16)]
                return carry

            lax.fori_loop(0, _CH, _fill, 0)

        def start_gathers(sl):
            return [
                pltpu.async_copy(x_hbm.at[sidx[sl]],
                                 rowbuf[sl].at[:, pl.ds(0, _D)], gsem[sl]),
                pltpu.async_copy(x_hbm.at[eidx[sl]],
                                 rowbuf[sl].at[:, pl.ds(_D, _D)], gsem[sl]),
            ]

        def start_writes(cix, sl):
            r0 = chunk_row(cix)
            return [
                pltpu.async_copy(rowbuf[sl],
                                 out_hbm.at[b, pl.ds(r0, _CH), :], wsem[sl]),
            ]

        build_idx(0, 0)
        pend_g = [None, None]
        pend_w = [None, None]
        pend_g[0] = start_gathers(0)
        for cix in range(_NCHUNK):
            sl = cix & 1
            for d in pend_g[sl]:
                d.wait()
            pend_w[sl] = start_writes(cix, sl)
            if cix + 1 < _NCHUNK:
                nsl = 1 - sl
                if pend_w[nsl] is not None:
                    for d in pend_w[nsl]:
                        d.wait()
                    pend_w[nsl] = None
                build_idx(cix + 1, nsl)
                pend_g[nsl] = start_gathers(nsl)
        for sl in (0, 1):
            if pend_w[sl] is not None:
                for d in pend_w[sl]:
                    d.wait()


def _span_index_table():
    starts_list, ends_list = [], []
    for w in range(1, _SPAN_MAX_LEN + 1):
        st = np.arange(0, _S - w + 1, dtype=np.int32)
        starts_list.append(st)
        ends_list.append(st + w - 1)
    return np.concatenate(starts_list), np.concatenate(ends_list)


_STARTS_NP, _ENDS_NP = _span_index_table()


def kernel(x, emb_table, batch_max_seq_len):
    mesh = plsc.VectorSubcoreMesh(core_axis_name="c", subcore_axis_name="s")
    out = pl.kernel(
        _body,
        mesh=mesh,
        out_type=jax.ShapeDtypeStruct((_B, _N, _ROW), jnp.float32),
        scratch_types=[
            [pltpu.VMEM((_CH, _ROW), jnp.float32)] * 2,
            [pltpu.VMEM((_CH,), jnp.int32)] * 2,
            [pltpu.VMEM((_CH,), jnp.int32)] * 2,
            pltpu.VMEM((len(_BINS), _E), jnp.float32),
            [pltpu.SemaphoreType.DMA] * 2,
            [pltpu.SemaphoreType.DMA] * 2,
        ],
    )(x.reshape(_B * _S, _D), emb_table)

    starts_j = jnp.asarray(_STARTS_NP)
    ends_j = jnp.minimum(jnp.asarray(_ENDS_NP), batch_max_seq_len - 1)
    span_indices = jnp.stack([starts_j, ends_j], axis=1)
    return out, span_indices
